# Initial kernel scaffold; baseline (speedup 1.0000x reference)
import functools

import jax
import jax.numpy as jnp
from jax import lax
from jax.experimental import pallas as pl
from jax.experimental.pallas import tpu as pltpu
from jax.experimental.pallas import tpu_sc as plsc

N = 10000
E = 320000
D = 64
NC, NS, L = 2, 16, 16
NW = NC * NS
K = 128
EPT = E // NW
C = -(-EPT // K)
EPT_PAD = C * K
N_PAD = 10240
RPT = N_PAD // NS
BN = 512
EPS = 1e-5

_mesh = plsc.VectorSubcoreMesh(core_axis_name="c", subcore_axis_name="s")


def _sc_body(with_deg, *refs):
    if with_deg:
        (z_hbm, srcs_hbm, dsts_hbm, zrow_hbm, ones_hbm, zrow16_hbm,
         out_hbm, deg_hbm, src_v, dst_v, rows_v, ones_v, acc_sh, deg_sh,
         sem) = refs
    else:
        (z_hbm, srcs_hbm, dsts_hbm, zrow_hbm,
         out_hbm, src_v, dst_v, rows_v, acc_sh, sem) = refs

    cid = lax.axis_index("c")
    sid = lax.axis_index("s")
    wid = cid * NS + sid

    pltpu.sync_copy(srcs_hbm.at[wid], src_v)
    pltpu.sync_copy(dsts_hbm.at[wid], dst_v)

    pltpu.sync_copy(zrow_hbm, rows_v)
    for j in range(RPT // K):
        pltpu.sync_copy(rows_v, acc_sh.at[pl.ds(sid * RPT + j * K, K)])
    if with_deg:
        pltpu.sync_copy(zrow16_hbm, ones_v)
        for j in range(RPT // K):
            pltpu.sync_copy(ones_v, deg_sh.at[pl.ds(sid * RPT + j * K, K)])
        pltpu.sync_copy(ones_hbm, ones_v)
    plsc.subcore_barrier()

    def body(c, carry):
        pltpu.async_copy(z_hbm.at[src_v.at[c]], rows_v, sem).wait()
        pltpu.sync_copy(rows_v, acc_sh.at[dst_v.at[c]], add=True)
        if with_deg:
            pltpu.sync_copy(ones_v, deg_sh.at[dst_v.at[c]], add=True)
        return carry

    lax.fori_loop(0, C, body, 0)
    plsc.subcore_barrier()

    sl = pl.ds(sid * RPT, RPT)
    pltpu.sync_copy(acc_sh.at[sl], out_hbm.at[cid, sl])
    if with_deg:
        pltpu.sync_copy(deg_sh.at[sl], deg_hbm.at[cid, sl])


_sc_agg = functools.partial(
    pl.kernel,
    functools.partial(_sc_body, False),
    out_type=jax.ShapeDtypeStruct((NC, N_PAD, D), jnp.float32),
    mesh=_mesh,
    scratch_types=[
        pltpu.VMEM((C, K), jnp.int32),
        pltpu.VMEM((C, K), jnp.int32),
        pltpu.VMEM((K, D), jnp.float32),
        pltpu.VMEM_SHARED((N_PAD, D), jnp.float32),
        pltpu.SemaphoreType.DMA,
    ],
)()

_sc_agg_deg = functools.partial(
    pl.kernel,
    functools.partial(_sc_body, True),
    out_type=(jax.ShapeDtypeStruct((NC, N_PAD, D), jnp.float32),
              jax.ShapeDtypeStruct((NC, N_PAD, L), jnp.float32)),
    mesh=_mesh,
    scratch_types=[
        pltpu.VMEM((C, K), jnp.int32),
        pltpu.VMEM((C, K), jnp.int32),
        pltpu.VMEM((K, D), jnp.float32),
        pltpu.VMEM((K, L), jnp.float32),
        pltpu.VMEM_SHARED((N_PAD, D), jnp.float32),
        pltpu.VMEM_SHARED((N_PAD, L), jnp.float32),
        pltpu.SemaphoreType.DMA,
    ],
)()


def _mm(a, b):
    return jnp.dot(a, b, preferred_element_type=jnp.float32)


def _tc_pre_body(x_ref, w_ref, o_ref):
    o_ref[...] = _mm(x_ref[...], w_ref[...])


def _layer_tail(pre, g, beta):
    mu = jnp.mean(pre, axis=-1, keepdims=True)
    var = jnp.mean((pre - mu) ** 2, axis=-1, keepdims=True)
    h = (pre - mu) / jnp.sqrt(var + EPS) * g + beta
    return jnp.maximum(h, 0.0)


def _tc_stage_body(has_next, p_ref, dp_ref, h_ref, wr_ref, b_ref, g_ref,
                   beta_ref, *rest):
    if has_next:
        wl_ref, ho_ref, zo_ref = rest
    else:
        (ho_ref,) = rest
    dp = dp_ref[...]
    deg = dp[0, :, 0] + dp[1, :, 0]
    agg = (p_ref[0] + p_ref[1]) / jnp.maximum(deg, 1.0)[:, None]
    pre = agg + b_ref[...] + _mm(h_ref[...], wr_ref[...])
    h = _layer_tail(pre, g_ref[...], beta_ref[...])
    ho_ref[...] = h
    if has_next:
        zo_ref[...] = _mm(h, wl_ref[...])


def _tc_final_body(p_ref, dp_ref, h_ref, wl_ref, b_ref, wr_ref, g_ref,
                   beta_ref, o_ref):
    dp = dp_ref[...]
    deg = dp[0, :, 0] + dp[1, :, 0]
    agg = (p_ref[0] + p_ref[1]) / jnp.maximum(deg, 1.0)[:, None]
    pre = _mm(agg, wl_ref[...]) + b_ref[...] + _mm(h_ref[...], wr_ref[...])
    o_ref[...] = _layer_tail(pre, g_ref[...], beta_ref[...])


def _row_spec(d):
    return pl.BlockSpec((BN, d), lambda i: (i, 0))


def _full_spec(*shape):
    return pl.BlockSpec(shape, lambda i: (0,) * len(shape))


_GRID = (N_PAD // BN,)

_P_SPEC = pl.BlockSpec((NC, BN, D), lambda i: (0, i, 0))
_DP_SPEC = pl.BlockSpec((NC, BN, L), lambda i: (0, i, 0))


def _tc_pre():
    return pl.pallas_call(
        _tc_pre_body,
        grid=_GRID,
        in_specs=[_row_spec(128), _full_spec(128, D)],
        out_specs=_row_spec(D),
        out_shape=jax.ShapeDtypeStruct((N_PAD, D), jnp.float32),
    )


def _tc_stage(has_next):
    in_specs = [_P_SPEC, _DP_SPEC, _row_spec(D), _full_spec(D, D),
                _full_spec(1, D), _full_spec(1, D), _full_spec(1, D)]
    if has_next:
        in_specs.append(_full_spec(D, D))
        out_specs = [_row_spec(D), _row_spec(D)]
        out_shape = [jax.ShapeDtypeStruct((N_PAD, D), jnp.float32)] * 2
    else:
        out_specs = _row_spec(D)
        out_shape = jax.ShapeDtypeStruct((N_PAD, D), jnp.float32)
    return pl.pallas_call(
        functools.partial(_tc_stage_body, has_next),
        grid=_GRID,
        in_specs=in_specs,
        out_specs=out_specs,
        out_shape=out_shape,
    )


def _tc_stage0():
    in_specs = [_P_SPEC, _DP_SPEC, _row_spec(128), _full_spec(128, D),
                _full_spec(1, D), _full_spec(1, D), _full_spec(1, D),
                _full_spec(D, D)]
    return pl.pallas_call(
        functools.partial(_tc_stage_body, True),
        grid=_GRID,
        in_specs=in_specs,
        out_specs=[_row_spec(D), _row_spec(D)],
        out_shape=[jax.ShapeDtypeStruct((N_PAD, D), jnp.float32)] * 2,
    )


def _tc_final():
    in_specs = [_P_SPEC, _DP_SPEC, _row_spec(D), _full_spec(D, 128),
                _full_spec(1, 128), _full_spec(D, 128), _full_spec(1, 128),
                _full_spec(1, 128)]
    return pl.pallas_call(
        _tc_final_body,
        grid=_GRID,
        in_specs=in_specs,
        out_specs=_row_spec(128),
        out_shape=jax.ShapeDtypeStruct((N_PAD, 128), jnp.float32),
    )


def kernel(x, edge_index, Wl0, bl0, Wr0, g0, beta0, Wl1, bl1, Wr1, g1, beta1,
           Wl2, bl2, Wr2, g2, beta2, Wl3, bl3, Wr3, g3, beta3):
    f32 = jnp.float32
    src = edge_index[0].astype(jnp.int32)
    dst = edge_index[1].astype(jnp.int32)
    pad_e = NW * EPT_PAD - E
    srcs = jnp.concatenate([src, jnp.zeros((pad_e,), jnp.int32)]).reshape(NW, C, K)
    dsts = jnp.concatenate([dst, jnp.full((pad_e,), N, jnp.int32)]).reshape(NW, C, K)
    x_pad = jnp.zeros((N_PAD, 128), f32).at[:N].set(x)
    zrow = jnp.zeros((K, D), f32)
    zrow16 = jnp.zeros((K, L), f32)
    ones16 = jnp.ones((K, L), f32)

    r1 = lambda v: v.reshape(1, -1).astype(f32)

    z0 = _tc_pre()(x_pad, Wl0)
    p0, dp = _sc_agg_deg(z0, srcs, dsts, zrow, ones16, zrow16)
    h1, z1 = _tc_stage0()(p0, dp, x_pad, Wr0, r1(bl0), r1(g0), r1(beta0), Wl1)
    p1 = _sc_agg(z1, srcs, dsts, zrow)
    h2, z2 = _tc_stage(True)(p1, dp, h1, Wr1, r1(bl1), r1(g1), r1(beta1), Wl2)
    p2 = _sc_agg(z2, srcs, dsts, zrow)
    h3 = _tc_stage(False)(p2, dp, h2, Wr2, r1(bl2), r1(g2), r1(beta2))
    p3 = _sc_agg(h3, srcs, dsts, zrow)
    out = _tc_final()(p3, dp, h3, Wl3, r1(bl3), Wr3, r1(g3), r1(beta3))
    return out[:N]


# speedup vs baseline: 6.6839x; 6.6839x over previous
import functools

import jax
import jax.numpy as jnp
from jax import lax
from jax.experimental import pallas as pl
from jax.experimental.pallas import tpu as pltpu
from jax.experimental.pallas import tpu_sc as plsc

N = 10000
E = 320000
D = 64
NC, NS, L = 2, 16, 16
NW = NC * NS
K = 128
EPT = E // NW
C = -(-EPT // K)
EPT_PAD = C * K
N_PAD = 10240
RPT = N_PAD // NS
BN = 512
EPS = 1e-5

_mesh = plsc.VectorSubcoreMesh(core_axis_name="c", subcore_axis_name="s")
_sc_params = pltpu.CompilerParams(use_tc_tiling_on_sc=False)


def _sc_body(with_deg, *refs):
    if with_deg:
        (z_hbm, srcs_hbm, dsts_hbm, zrow_hbm, ones_hbm, zrow16_hbm,
         out_hbm, deg_hbm, src_v, dst_v, rows_v, ones_v, acc_sh, deg_sh,
         sem) = refs
    else:
        (z_hbm, srcs_hbm, dsts_hbm, zrow_hbm,
         out_hbm, src_v, dst_v, rows_v, acc_sh, sem) = refs

    cid = lax.axis_index("c")
    sid = lax.axis_index("s")
    wid = cid * NS + sid

    pltpu.sync_copy(srcs_hbm.at[wid], src_v)
    pltpu.sync_copy(dsts_hbm.at[wid], dst_v)

    pltpu.sync_copy(zrow_hbm, rows_v)
    for j in range(RPT // K):
        pltpu.sync_copy(rows_v, acc_sh.at[pl.ds(sid * RPT + j * K, K)])
    if with_deg:
        pltpu.sync_copy(zrow16_hbm, ones_v)
        for j in range(RPT // K):
            pltpu.sync_copy(ones_v, deg_sh.at[pl.ds(sid * RPT + j * K, K)])
        pltpu.sync_copy(ones_hbm, ones_v)
    plsc.subcore_barrier()

    def body(c, carry):
        pltpu.async_copy(z_hbm.at[src_v.at[c]], rows_v, sem).wait()
        pltpu.sync_copy(rows_v, acc_sh.at[dst_v.at[c]], add=True)
        if with_deg:
            pltpu.sync_copy(ones_v, deg_sh.at[dst_v.at[c]], add=True)
        return carry

    lax.fori_loop(0, C, body, 0)
    plsc.subcore_barrier()

    sl = pl.ds(sid * RPT, RPT)
    pltpu.sync_copy(acc_sh.at[sl], out_hbm.at[cid, sl])
    if with_deg:
        pltpu.sync_copy(deg_sh.at[sl], deg_hbm.at[cid, sl])


_sc_agg = functools.partial(
    pl.kernel,
    functools.partial(_sc_body, False),
    out_type=jax.ShapeDtypeStruct((NC, N_PAD, D), jnp.float32),
    mesh=_mesh,
    scratch_types=[
        pltpu.VMEM((C, K), jnp.int32),
        pltpu.VMEM((C, K), jnp.int32),
        pltpu.VMEM((K, D), jnp.float32),
        pltpu.VMEM_SHARED((N_PAD, D), jnp.float32),
        pltpu.SemaphoreType.DMA,
    ],
    compiler_params=_sc_params,
)()

_sc_agg_deg = functools.partial(
    pl.kernel,
    functools.partial(_sc_body, True),
    out_type=(jax.ShapeDtypeStruct((NC, N_PAD, D), jnp.float32),
              jax.ShapeDtypeStruct((NC, N_PAD, L), jnp.float32)),
    mesh=_mesh,
    scratch_types=[
        pltpu.VMEM((C, K), jnp.int32),
        pltpu.VMEM((C, K), jnp.int32),
        pltpu.VMEM((K, D), jnp.float32),
        pltpu.VMEM((K, L), jnp.float32),
        pltpu.VMEM_SHARED((N_PAD, D), jnp.float32),
        pltpu.VMEM_SHARED((N_PAD, L), jnp.float32),
        pltpu.SemaphoreType.DMA,
    ],
    compiler_params=_sc_params,
)()


def _mm(a, b):
    return jnp.dot(a, b, preferred_element_type=jnp.float32)


def _tc_pre_body(x_ref, w_ref, o_ref):
    o_ref[...] = _mm(x_ref[...], w_ref[...])


def _layer_tail(pre, g, beta):
    mu = jnp.mean(pre, axis=-1, keepdims=True)
    var = jnp.mean((pre - mu) ** 2, axis=-1, keepdims=True)
    h = (pre - mu) / jnp.sqrt(var + EPS) * g + beta
    return jnp.maximum(h, 0.0)


def _tc_stage_body(has_next, p_ref, dp_ref, h_ref, wr_ref, b_ref, g_ref,
                   beta_ref, *rest):
    if has_next:
        wl_ref, ho_ref, zo_ref = rest
    else:
        (ho_ref,) = rest
    dp = dp_ref[...]
    deg = dp[0, :, 0] + dp[1, :, 0]
    agg = (p_ref[0] + p_ref[1]) / jnp.maximum(deg, 1.0)[:, None]
    pre = agg + b_ref[...] + _mm(h_ref[...], wr_ref[...])
    h = _layer_tail(pre, g_ref[...], beta_ref[...])
    ho_ref[...] = h
    if has_next:
        zo_ref[...] = _mm(h, wl_ref[...])


def _tc_final_body(p_ref, dp_ref, h_ref, wl_ref, b_ref, wr_ref, g_ref,
                   beta_ref, o_ref):
    dp = dp_ref[...]
    deg = dp[0, :, 0] + dp[1, :, 0]
    agg = (p_ref[0] + p_ref[1]) / jnp.maximum(deg, 1.0)[:, None]
    pre = _mm(agg, wl_ref[...]) + b_ref[...] + _mm(h_ref[...], wr_ref[...])
    o_ref[...] = _layer_tail(pre, g_ref[...], beta_ref[...])


def _row_spec(d):
    return pl.BlockSpec((BN, d), lambda i: (i, 0))


def _full_spec(*shape):
    return pl.BlockSpec(shape, lambda i: (0,) * len(shape))


_GRID = (N_PAD // BN,)

_P_SPEC = pl.BlockSpec((NC, BN, D), lambda i: (0, i, 0))
_DP_SPEC = pl.BlockSpec((NC, BN, L), lambda i: (0, i, 0))


def _tc_pre():
    return pl.pallas_call(
        _tc_pre_body,
        grid=_GRID,
        in_specs=[_row_spec(128), _full_spec(128, D)],
        out_specs=_row_spec(D),
        out_shape=jax.ShapeDtypeStruct((N_PAD, D), jnp.float32),
    )


def _tc_stage(has_next):
    in_specs = [_P_SPEC, _DP_SPEC, _row_spec(D), _full_spec(D, D),
                _full_spec(1, D), _full_spec(1, D), _full_spec(1, D)]
    if has_next:
        in_specs.append(_full_spec(D, D))
        out_specs = [_row_spec(D), _row_spec(D)]
        out_shape = [jax.ShapeDtypeStruct((N_PAD, D), jnp.float32)] * 2
    else:
        out_specs = _row_spec(D)
        out_shape = jax.ShapeDtypeStruct((N_PAD, D), jnp.float32)
    return pl.pallas_call(
        functools.partial(_tc_stage_body, has_next),
        grid=_GRID,
        in_specs=in_specs,
        out_specs=out_specs,
        out_shape=out_shape,
    )


def _tc_stage0():
    in_specs = [_P_SPEC, _DP_SPEC, _row_spec(128), _full_spec(128, D),
                _full_spec(1, D), _full_spec(1, D), _full_spec(1, D),
                _full_spec(D, D)]
    return pl.pallas_call(
        functools.partial(_tc_stage_body, True),
        grid=_GRID,
        in_specs=in_specs,
        out_specs=[_row_spec(D), _row_spec(D)],
        out_shape=[jax.ShapeDtypeStruct((N_PAD, D), jnp.float32)] * 2,
    )


def _tc_final():
    in_specs = [_P_SPEC, _DP_SPEC, _row_spec(D), _full_spec(D, 128),
                _full_spec(1, 128), _full_spec(D, 128), _full_spec(1, 128),
                _full_spec(1, 128)]
    return pl.pallas_call(
        _tc_final_body,
        grid=_GRID,
        in_specs=in_specs,
        out_specs=_row_spec(128),
        out_shape=jax.ShapeDtypeStruct((N_PAD, 128), jnp.float32),
    )


def kernel(x, edge_index, Wl0, bl0, Wr0, g0, beta0, Wl1, bl1, Wr1, g1, beta1,
           Wl2, bl2, Wr2, g2, beta2, Wl3, bl3, Wr3, g3, beta3):
    f32 = jnp.float32
    src = edge_index[0].astype(jnp.int32)
    dst = edge_index[1].astype(jnp.int32)
    pad_e = NW * EPT_PAD - E
    srcs = jnp.concatenate([src, jnp.zeros((pad_e,), jnp.int32)]).reshape(NW, C, K)
    dsts = jnp.concatenate([dst, jnp.full((pad_e,), N, jnp.int32)]).reshape(NW, C, K)
    x_pad = jnp.zeros((N_PAD, 128), f32).at[:N].set(x)
    zrow = jnp.zeros((K, D), f32)
    zrow16 = jnp.zeros((K, L), f32)
    ones16 = jnp.ones((K, L), f32)

    r1 = lambda v: v.reshape(1, -1).astype(f32)

    z0 = _tc_pre()(x_pad, Wl0)
    p0, dp = _sc_agg_deg(z0, srcs, dsts, zrow, ones16, zrow16)
    h1, z1 = _tc_stage0()(p0, dp, x_pad, Wr0, r1(bl0), r1(g0), r1(beta0), Wl1)
    p1 = _sc_agg(z1, srcs, dsts, zrow)
    h2, z2 = _tc_stage(True)(p1, dp, h1, Wr1, r1(bl1), r1(g1), r1(beta1), Wl2)
    p2 = _sc_agg(z2, srcs, dsts, zrow)
    h3 = _tc_stage(False)(p2, dp, h2, Wr2, r1(bl2), r1(g2), r1(beta2))
    p3 = _sc_agg(h3, srcs, dsts, zrow)
    out = _tc_final()(p3, dp, h3, Wl3, r1(bl3), Wr3, r1(g3), r1(beta3))
    return out[:N]
